# baseline (speedup 1.0000x reference)
import functools

import jax
import jax.numpy as jnp
from jax import lax
from jax.experimental import pallas as pl
from jax.experimental.pallas import tpu as pltpu
from jax.experimental.pallas import tpu_sc as plsc

N_NODES = 10000
N_GRAPHS = 64
D = 128

NPAD = 10240
EPAD = 327680
K = 128
CHUNKS_PER_TILE = 80
ROWS_PER_TILE = 640
GPAD = 128
BCHUNKS = 80
PAD_DST = NPAD - 8
PAD_GRAPH = N_GRAPHS
CW = 16

_MESH = plsc.VectorSubcoreMesh(core_axis_name="c", subcore_axis_name="s")
_SC_PARAMS = pltpu.CompilerParams(use_tc_tiling_on_sc=False)


def _zero_my_rows(zrows_hbm, rows_v, table_s, r0, nrows):
    pltpu.sync_copy(zrows_hbm, rows_v)
    @pl.loop(0, nrows // K)
    def _(j):
        pltpu.sync_copy(rows_v, table_s.at[pl.ds(r0 + j * K, K)])


_G = 16
_NG = [10, 0]
assert 16 * (_NG[0] + _NG[1]) * _G == EPAD // K


def _make_agg_kernel():
    outs = [jax.ShapeDtypeStruct((2 * NPAD, D), jnp.float32)]
    scratch = [
        pltpu.VMEM((_G, K), jnp.int32),
        pltpu.VMEM((_G, K), jnp.int32),
        pltpu.VMEM((K, D), jnp.float32),
        pltpu.VMEM((K, D), jnp.float32),
        pltpu.VMEM_SHARED((NPAD, D), jnp.float32),
        pltpu.SemaphoreType.DMA,
        pltpu.SemaphoreType.DMA,
        pltpu.SemaphoreType.DMA,
        pltpu.SemaphoreType.DMA,
    ]

    def body(h_hbm, src_hbm, dst_hbm, zrows_hbm, agg_hbm,
             src_v, dst_v, rows0, rows1, agg_s, g0, g1, s0, s1):
        cid = lax.axis_index("c")
        sid = lax.axis_index("s")
        r0 = sid * ROWS_PER_TILE

        _zero_my_rows(zrows_hbm, rows0, agg_s, r0, ROWS_PER_TILE)
        ngroups = jnp.where(cid == 0, _NG[0], _NG[1])
        c0 = jnp.where(cid == 0, sid * (_NG[0] * _G),
                       16 * _NG[0] * _G + sid * (_NG[1] * _G))
        plsc.subcore_barrier()

        def gath(i, buf, sem):
            pltpu.async_copy(h_hbm.at[src_v.at[i]], buf, sem)

        def wait_g(i, buf, sem):
            pltpu.make_async_copy(h_hbm.at[src_v.at[i]], buf, sem).wait()

        def scat(i, buf, sem):
            pltpu.async_copy(buf, agg_s.at[dst_v.at[i]], sem, add=True)

        def wait_s(i, buf, sem):
            pltpu.make_async_copy(
                buf, agg_s.at[dst_v.at[i]], sem).wait()

        @pl.loop(0, ngroups)
        def _(grp):
            h0 = c0 + grp * _G
            pltpu.sync_copy(src_hbm.at[pl.ds(h0, _G)], src_v)
            pltpu.sync_copy(dst_hbm.at[pl.ds(h0, _G)], dst_v)
            gath(0, rows0, g0)
            @pl.loop(0, _G // 2)
            def _(j):
                i0 = 2 * j
                i1 = 2 * j + 1
                wait_g(i0, rows0, g0)
                scat(i0, rows0, s0)
                @pl.when(j > 0)
                def _():
                    wait_s(i1, rows1, s1)
                gath(i1, rows1, g1)
                wait_g(i1, rows1, g1)
                scat(i1, rows1, s1)
                wait_s(i0, rows0, s0)
                @pl.when(j < _G // 2 - 1)
                def _():
                    gath(i0 + 2, rows0, g0)
            wait_s(_G - 1, rows1, s1)

        plsc.subcore_barrier()
        out0 = cid * NPAD + r0
        @pl.loop(0, ROWS_PER_TILE // K)
        def _(j):
            pltpu.sync_copy(agg_s.at[pl.ds(r0 + j * K, K)], rows0)
            pltpu.sync_copy(rows0, agg_hbm.at[pl.ds(out0 + j * K, K)])

    return pl.kernel(body, out_type=outs, mesh=_MESH, scratch_types=scratch,
                     compiler_params=_SC_PARAMS, name="sc_agg")


_agg_kernel = _make_agg_kernel()


_HALF = CHUNKS_PER_TILE // 2


def _cnt_body(dst_hbm, ones_hbm, zcol_hbm, cnt_hbm,
              dst_v, ones_v, col_v, cnt_s):
    cid = lax.axis_index("c")
    sid = lax.axis_index("s")
    wid = sid * 2 + cid
    r0 = sid * ROWS_PER_TILE

    pltpu.sync_copy(zcol_hbm, col_v)
    @pl.loop(0, ROWS_PER_TILE // K)
    def _(j):
        pltpu.sync_copy(col_v, cnt_s.at[pl.ds(r0 + j * K, K)])
    pltpu.sync_copy(ones_hbm, ones_v)
    c0 = wid * CHUNKS_PER_TILE
    plsc.subcore_barrier()

    @pl.loop(0, 2)
    def _(half):
        pltpu.sync_copy(dst_hbm.at[pl.ds(c0 + half * _HALF, _HALF)], dst_v)
        @pl.loop(0, _HALF)
        def _(i):
            pltpu.sync_copy(ones_v, cnt_s.at[dst_v.at[i]], add=True)

    plsc.subcore_barrier()
    out0 = cid * NPAD + r0
    @pl.loop(0, ROWS_PER_TILE // K)
    def _(j):
        pltpu.sync_copy(cnt_s.at[pl.ds(r0 + j * K, K)], col_v)
        pltpu.sync_copy(col_v, cnt_hbm.at[pl.ds(out0 + j * K, K)])


_cnt_kernel = pl.kernel(
    _cnt_body,
    out_type=[jax.ShapeDtypeStruct((2 * NPAD, CW), jnp.float32)],
    mesh=_MESH,
    scratch_types=[
        pltpu.VMEM((_HALF, K), jnp.int32),
        pltpu.VMEM((K, CW), jnp.float32),
        pltpu.VMEM((K, CW), jnp.float32),
        pltpu.VMEM_SHARED((NPAD, CW), jnp.float32),
    ],
    compiler_params=_SC_PARAMS,
    name="sc_cnt",
)


def _pool_body(h_hbm, bidx_hbm, ones_hbm, zrows_hbm, zcol_hbm,
               pool_hbm, pcnt_hbm, bidx_v, rows_v, ones_v, col_v,
               pool_s, pcnt_s, sem):
    cid = lax.axis_index("c")
    sid = lax.axis_index("s")
    wid = sid * 2 + cid

    @pl.when(sid == 0)
    def _():
        _zero_my_rows(zrows_hbm, rows_v, pool_s, 0, GPAD)
        pltpu.sync_copy(zcol_hbm, col_v)
        pltpu.sync_copy(col_v, pcnt_s)
    pltpu.sync_copy(ones_hbm, ones_v)
    pltpu.sync_copy(bidx_hbm.at[pl.ds(0, BCHUNKS)], bidx_v)
    plsc.subcore_barrier()

    @pl.loop(0, 3)
    def _(j):
        chunk = wid + j * 32
        @pl.when(chunk < BCHUNKS)
        def _():
            pltpu.async_copy(h_hbm.at[pl.ds(chunk * K, K)], rows_v, sem).wait()
            pltpu.sync_copy(rows_v, pool_s.at[bidx_v.at[chunk]], add=True)
            pltpu.sync_copy(ones_v, pcnt_s.at[bidx_v.at[chunk]], add=True)

    plsc.subcore_barrier()
    @pl.when(sid == 0)
    def _():
        pltpu.sync_copy(pool_s, rows_v)
        pltpu.sync_copy(rows_v, pool_hbm.at[pl.ds(cid * GPAD, GPAD)])
        pltpu.sync_copy(pcnt_s, col_v)
        pltpu.sync_copy(col_v, pcnt_hbm.at[pl.ds(cid * GPAD, GPAD)])


_pool_kernel = pl.kernel(
    _pool_body,
    out_type=[jax.ShapeDtypeStruct((2 * GPAD, D), jnp.float32),
              jax.ShapeDtypeStruct((2 * GPAD, CW), jnp.float32)],
    mesh=_MESH,
    scratch_types=[
        pltpu.VMEM((BCHUNKS, K), jnp.int32),
        pltpu.VMEM((K, D), jnp.float32),
        pltpu.VMEM((K, CW), jnp.float32),
        pltpu.VMEM((GPAD, CW), jnp.float32),
        pltpu.VMEM_SHARED((GPAD, D), jnp.float32),
        pltpu.VMEM_SHARED((GPAD, CW), jnp.float32),
        pltpu.SemaphoreType.DMA,
    ],
    compiler_params=_SC_PARAMS,
    name="sc_pool",
)


_DOT = functools.partial(
    lax.dot_general,
    precision=lax.Precision.HIGHEST,
    preferred_element_type=jnp.float32,
)

_RBLK = 640
_RGRID = NPAD // _RBLK


def _dense_body(relu, aggA, aggB, cA, cB, h, Wl, Wr, bl, o):
    cnt = jnp.maximum(cA[...] + cB[...], 1.0)[:, :1]
    mean = (aggA[...] + aggB[...]) / cnt
    r = _DOT(mean, Wl[...], (((1,), (1,)), ((), ())))
    r = r + _DOT(h[...], Wr[...], (((1,), (1,)), ((), ())))
    r = r + bl[...]
    o[...] = jnp.maximum(r, 0.0) if relu else r


def _dense_layer(agg, cnt, h, Wl, bl, Wr, relu):
    body = functools.partial(_dense_body, relu)
    return pl.pallas_call(
        body,
        grid=(_RGRID,),
        in_specs=[
            pl.BlockSpec((_RBLK, D), lambda i: (i, 0)),
            pl.BlockSpec((_RBLK, D), lambda i: (i + _RGRID, 0)),
            pl.BlockSpec((_RBLK, CW), lambda i: (i, 0)),
            pl.BlockSpec((_RBLK, CW), lambda i: (i + _RGRID, 0)),
            pl.BlockSpec((_RBLK, D), lambda i: (i, 0)),
            pl.BlockSpec((D, D), lambda i: (0, 0)),
            pl.BlockSpec((D, D), lambda i: (0, 0)),
            pl.BlockSpec((1, D), lambda i: (0, 0)),
        ],
        out_specs=pl.BlockSpec((_RBLK, D), lambda i: (i, 0)),
        out_shape=jax.ShapeDtypeStruct((NPAD, D), jnp.float32),
    )(agg, agg, cnt, cnt, h, Wl, Wr, bl)


def _final_body(pA, pB, cA, cB, W, b, o):
    cnt = jnp.maximum(cA[...] + cB[...], 1.0)[:, :1]
    g = (pA[...] + pB[...]) / cnt
    o[...] = _DOT(g, W[...], (((1,), (1,)), ((), ()))) + b[...]


def _final_layer(pool, pcnt, W, b):
    nb = GPAD // N_GRAPHS
    return pl.pallas_call(
        _final_body,
        grid=(1,),
        in_specs=[
            pl.BlockSpec((N_GRAPHS, D), lambda i: (0, 0)),
            pl.BlockSpec((N_GRAPHS, D), lambda i: (nb, 0)),
            pl.BlockSpec((N_GRAPHS, CW), lambda i: (0, 0)),
            pl.BlockSpec((N_GRAPHS, CW), lambda i: (nb, 0)),
            pl.BlockSpec(W.shape, lambda i: (0, 0)),
            pl.BlockSpec((1, W.shape[0]), lambda i: (0, 0)),
        ],
        out_specs=pl.BlockSpec((N_GRAPHS, W.shape[0]), lambda i: (0, 0)),
        out_shape=jax.ShapeDtypeStruct((N_GRAPHS, W.shape[0]), jnp.float32),
    )(pool, pool, pcnt, pcnt, W, b)


def kernel(x, edge_index, batch, Wl1, bl1, Wr1, Wl2, bl2, Wr2, Wl3, bl3,
           Wr3, W, b):
    e = edge_index.shape[1]
    src = jnp.asarray(edge_index[0], jnp.int32)
    dst = jnp.asarray(edge_index[1], jnp.int32)
    src = jnp.concatenate(
        [src, jnp.zeros((EPAD - e,), jnp.int32)]).reshape(-1, K)
    dst = jnp.concatenate(
        [dst, jnp.full((EPAD - e,), PAD_DST, jnp.int32)]).reshape(-1, K)
    bidx = jnp.concatenate(
        [jnp.asarray(batch, jnp.int32),
         jnp.full((NPAD - N_NODES,), PAD_GRAPH, jnp.int32)]).reshape(-1, K)

    xpad = jnp.concatenate(
        [x, jnp.zeros((NPAD - N_NODES, D), jnp.float32)], axis=0)
    ones = jnp.ones((K, CW), jnp.float32)
    zrows = jnp.zeros((K, D), jnp.float32)
    zcol = jnp.zeros((K, CW), jnp.float32)
    zcol_g = jnp.zeros((GPAD, CW), jnp.float32)
    bl1r, bl2r, bl3r = (v.reshape(1, D) for v in (bl1, bl2, bl3))

    cnt, = _cnt_kernel(dst, ones, zcol)
    agg1, = _agg_kernel(xpad, src, dst, zrows)
    h1 = _dense_layer(agg1, cnt, xpad, Wl1, bl1r, Wr1, relu=True)
    agg2, = _agg_kernel(h1, src, dst, zrows)
    h2 = _dense_layer(agg2, cnt, h1, Wl2, bl2r, Wr2, relu=True)
    agg3, = _agg_kernel(h2, src, dst, zrows)
    h3 = _dense_layer(agg3, cnt, h2, Wl3, bl3r, Wr3, relu=False)
    pool, pcnt = _pool_kernel(h3, bidx, ones, zrows, zcol_g)
    return _final_layer(pool, pcnt, W, b.reshape(1, -1))


# speedup vs baseline: 1.4651x; 1.4651x over previous
import functools

import jax
import jax.numpy as jnp
from jax import lax
from jax.experimental import pallas as pl
from jax.experimental.pallas import tpu as pltpu
from jax.experimental.pallas import tpu_sc as plsc

N_NODES = 10000
N_GRAPHS = 64
D = 128

NPAD = 10240
EPAD = 327680
K = 128
CHUNKS_PER_TILE = 80
ROWS_PER_TILE = 640
GPAD = 128
BCHUNKS = 80
PAD_DST = NPAD - 8
PAD_GRAPH = N_GRAPHS
CW = 16

_MESH = plsc.VectorSubcoreMesh(core_axis_name="c", subcore_axis_name="s")
_SC_PARAMS = pltpu.CompilerParams(use_tc_tiling_on_sc=False)


def _zero_my_rows(zrows_hbm, rows_v, table_s, r0, nrows):
    pltpu.sync_copy(zrows_hbm, rows_v)
    @pl.loop(0, nrows // K)
    def _(j):
        pltpu.sync_copy(rows_v, table_s.at[pl.ds(r0 + j * K, K)])


_G = 10
_NG = [14, 2]
assert 16 * (_NG[0] + _NG[1]) * _G == EPAD // K


def _make_agg_kernel():
    outs = [jax.ShapeDtypeStruct((2 * NPAD, D), jnp.float32)]
    scratch = [
        pltpu.VMEM((_G, K), jnp.int32),
        pltpu.VMEM((_G, K), jnp.int32),
        pltpu.VMEM((K, D), jnp.float32),
        pltpu.VMEM((K, D), jnp.float32),
        pltpu.VMEM_SHARED((NPAD, D), jnp.float32),
        pltpu.SemaphoreType.DMA,
        pltpu.SemaphoreType.DMA,
        pltpu.SemaphoreType.DMA,
        pltpu.SemaphoreType.DMA,
    ]

    def body(h_hbm, src_hbm, dst_hbm, zrows_hbm, agg_hbm,
             src_v, dst_v, rows0, rows1, agg_s, g0, g1, s0, s1):
        cid = lax.axis_index("c")
        sid = lax.axis_index("s")
        r0 = sid * ROWS_PER_TILE

        _zero_my_rows(zrows_hbm, rows0, agg_s, r0, ROWS_PER_TILE)
        ngroups = jnp.where(cid == 0, _NG[0], _NG[1])
        c0 = jnp.where(cid == 0, sid * (_NG[0] * _G),
                       16 * _NG[0] * _G + sid * (_NG[1] * _G))
        plsc.subcore_barrier()

        def gath(i, buf, sem):
            pltpu.async_copy(h_hbm.at[src_v.at[i]], buf, sem)

        def wait_g(i, buf, sem):
            pltpu.make_async_copy(h_hbm.at[src_v.at[i]], buf, sem).wait()

        def scat(i, buf, sem):
            pltpu.async_copy(buf, agg_s.at[dst_v.at[i]], sem, add=True)

        def wait_s(i, buf, sem):
            pltpu.make_async_copy(
                buf, agg_s.at[dst_v.at[i]], sem).wait()

        @pl.loop(0, ngroups)
        def _(grp):
            h0 = c0 + grp * _G
            pltpu.sync_copy(src_hbm.at[pl.ds(h0, _G)], src_v)
            pltpu.sync_copy(dst_hbm.at[pl.ds(h0, _G)], dst_v)
            gath(0, rows0, g0)
            @pl.loop(0, _G // 2)
            def _(j):
                i0 = 2 * j
                i1 = 2 * j + 1
                wait_g(i0, rows0, g0)
                scat(i0, rows0, s0)
                @pl.when(j > 0)
                def _():
                    wait_s(i1, rows1, s1)
                gath(i1, rows1, g1)
                wait_g(i1, rows1, g1)
                scat(i1, rows1, s1)
                wait_s(i0, rows0, s0)
                @pl.when(j < _G // 2 - 1)
                def _():
                    gath(i0 + 2, rows0, g0)
            wait_s(_G - 1, rows1, s1)

        plsc.subcore_barrier()
        out0 = cid * NPAD + r0
        @pl.loop(0, ROWS_PER_TILE // K)
        def _(j):
            pltpu.sync_copy(agg_s.at[pl.ds(r0 + j * K, K)], rows0)
            pltpu.sync_copy(rows0, agg_hbm.at[pl.ds(out0 + j * K, K)])

    return pl.kernel(body, out_type=outs, mesh=_MESH, scratch_types=scratch,
                     compiler_params=_SC_PARAMS, name="sc_agg")


_agg_kernel = _make_agg_kernel()


_HALF = CHUNKS_PER_TILE // 2


def _cnt_body(dst_hbm, ones_hbm, zcol_hbm, cnt_hbm,
              dst_v, ones_v, col_v, cnt_s):
    cid = lax.axis_index("c")
    sid = lax.axis_index("s")
    wid = sid * 2 + cid
    r0 = sid * ROWS_PER_TILE

    pltpu.sync_copy(zcol_hbm, col_v)
    @pl.loop(0, ROWS_PER_TILE // K)
    def _(j):
        pltpu.sync_copy(col_v, cnt_s.at[pl.ds(r0 + j * K, K)])
    pltpu.sync_copy(ones_hbm, ones_v)
    c0 = wid * CHUNKS_PER_TILE
    plsc.subcore_barrier()

    @pl.loop(0, 2)
    def _(half):
        pltpu.sync_copy(dst_hbm.at[pl.ds(c0 + half * _HALF, _HALF)], dst_v)
        @pl.loop(0, _HALF)
        def _(i):
            pltpu.sync_copy(ones_v, cnt_s.at[dst_v.at[i]], add=True)

    plsc.subcore_barrier()
    out0 = cid * NPAD + r0
    @pl.loop(0, ROWS_PER_TILE // K)
    def _(j):
        pltpu.sync_copy(cnt_s.at[pl.ds(r0 + j * K, K)], col_v)
        pltpu.sync_copy(col_v, cnt_hbm.at[pl.ds(out0 + j * K, K)])


_cnt_kernel = pl.kernel(
    _cnt_body,
    out_type=[jax.ShapeDtypeStruct((2 * NPAD, CW), jnp.float32)],
    mesh=_MESH,
    scratch_types=[
        pltpu.VMEM((_HALF, K), jnp.int32),
        pltpu.VMEM((K, CW), jnp.float32),
        pltpu.VMEM((K, CW), jnp.float32),
        pltpu.VMEM_SHARED((NPAD, CW), jnp.float32),
    ],
    compiler_params=_SC_PARAMS,
    name="sc_cnt",
)


def _pool_body(h_hbm, bidx_hbm, ones_hbm, zrows_hbm, zcol_hbm,
               pool_hbm, pcnt_hbm, bidx_v, rows_v, ones_v, col_v,
               pool_s, pcnt_s, sem):
    cid = lax.axis_index("c")
    sid = lax.axis_index("s")
    wid = sid * 2 + cid

    @pl.when(sid == 0)
    def _():
        _zero_my_rows(zrows_hbm, rows_v, pool_s, 0, GPAD)
        pltpu.sync_copy(zcol_hbm, col_v)
        pltpu.sync_copy(col_v, pcnt_s)
    pltpu.sync_copy(ones_hbm, ones_v)
    pltpu.sync_copy(bidx_hbm.at[pl.ds(0, BCHUNKS)], bidx_v)
    plsc.subcore_barrier()

    @pl.loop(0, 3)
    def _(j):
        chunk = wid + j * 32
        @pl.when(chunk < BCHUNKS)
        def _():
            pltpu.async_copy(h_hbm.at[pl.ds(chunk * K, K)], rows_v, sem).wait()
            pltpu.sync_copy(rows_v, pool_s.at[bidx_v.at[chunk]], add=True)
            pltpu.sync_copy(ones_v, pcnt_s.at[bidx_v.at[chunk]], add=True)

    plsc.subcore_barrier()
    @pl.when(sid == 0)
    def _():
        pltpu.sync_copy(pool_s, rows_v)
        pltpu.sync_copy(rows_v, pool_hbm.at[pl.ds(cid * GPAD, GPAD)])
        pltpu.sync_copy(pcnt_s, col_v)
        pltpu.sync_copy(col_v, pcnt_hbm.at[pl.ds(cid * GPAD, GPAD)])


_pool_kernel = pl.kernel(
    _pool_body,
    out_type=[jax.ShapeDtypeStruct((2 * GPAD, D), jnp.float32),
              jax.ShapeDtypeStruct((2 * GPAD, CW), jnp.float32)],
    mesh=_MESH,
    scratch_types=[
        pltpu.VMEM((BCHUNKS, K), jnp.int32),
        pltpu.VMEM((K, D), jnp.float32),
        pltpu.VMEM((K, CW), jnp.float32),
        pltpu.VMEM((GPAD, CW), jnp.float32),
        pltpu.VMEM_SHARED((GPAD, D), jnp.float32),
        pltpu.VMEM_SHARED((GPAD, CW), jnp.float32),
        pltpu.SemaphoreType.DMA,
    ],
    compiler_params=_SC_PARAMS,
    name="sc_pool",
)


_DOT = functools.partial(
    lax.dot_general,
    precision=lax.Precision.HIGHEST,
    preferred_element_type=jnp.float32,
)

_RBLK = 640
_RGRID = NPAD // _RBLK


def _dense_body(relu, aggA, aggB, cA, cB, h, Wl, Wr, bl, o):
    cnt = jnp.maximum(cA[...] + cB[...], 1.0)[:, :1]
    mean = (aggA[...] + aggB[...]) / cnt
    r = _DOT(mean, Wl[...], (((1,), (1,)), ((), ())))
    r = r + _DOT(h[...], Wr[...], (((1,), (1,)), ((), ())))
    r = r + bl[...]
    o[...] = jnp.maximum(r, 0.0) if relu else r


def _dense_layer(agg, cnt, h, Wl, bl, Wr, relu):
    body = functools.partial(_dense_body, relu)
    return pl.pallas_call(
        body,
        grid=(_RGRID,),
        in_specs=[
            pl.BlockSpec((_RBLK, D), lambda i: (i, 0)),
            pl.BlockSpec((_RBLK, D), lambda i: (i + _RGRID, 0)),
            pl.BlockSpec((_RBLK, CW), lambda i: (i, 0)),
            pl.BlockSpec((_RBLK, CW), lambda i: (i + _RGRID, 0)),
            pl.BlockSpec((_RBLK, D), lambda i: (i, 0)),
            pl.BlockSpec((D, D), lambda i: (0, 0)),
            pl.BlockSpec((D, D), lambda i: (0, 0)),
            pl.BlockSpec((1, D), lambda i: (0, 0)),
        ],
        out_specs=pl.BlockSpec((_RBLK, D), lambda i: (i, 0)),
        out_shape=jax.ShapeDtypeStruct((NPAD, D), jnp.float32),
    )(agg, agg, cnt, cnt, h, Wl, Wr, bl)


def _final_body(pA, pB, cA, cB, W, b, o):
    cnt = jnp.maximum(cA[...] + cB[...], 1.0)[:, :1]
    g = (pA[...] + pB[...]) / cnt
    o[...] = _DOT(g, W[...], (((1,), (1,)), ((), ()))) + b[...]


def _final_layer(pool, pcnt, W, b):
    nb = GPAD // N_GRAPHS
    return pl.pallas_call(
        _final_body,
        grid=(1,),
        in_specs=[
            pl.BlockSpec((N_GRAPHS, D), lambda i: (0, 0)),
            pl.BlockSpec((N_GRAPHS, D), lambda i: (nb, 0)),
            pl.BlockSpec((N_GRAPHS, CW), lambda i: (0, 0)),
            pl.BlockSpec((N_GRAPHS, CW), lambda i: (nb, 0)),
            pl.BlockSpec(W.shape, lambda i: (0, 0)),
            pl.BlockSpec((1, W.shape[0]), lambda i: (0, 0)),
        ],
        out_specs=pl.BlockSpec((N_GRAPHS, W.shape[0]), lambda i: (0, 0)),
        out_shape=jax.ShapeDtypeStruct((N_GRAPHS, W.shape[0]), jnp.float32),
    )(pool, pool, pcnt, pcnt, W, b)


def kernel(x, edge_index, batch, Wl1, bl1, Wr1, Wl2, bl2, Wr2, Wl3, bl3,
           Wr3, W, b):
    e = edge_index.shape[1]
    src = jnp.asarray(edge_index[0], jnp.int32)
    dst = jnp.asarray(edge_index[1], jnp.int32)
    src = jnp.concatenate(
        [src, jnp.zeros((EPAD - e,), jnp.int32)]).reshape(-1, K)
    dst = jnp.concatenate(
        [dst, jnp.full((EPAD - e,), PAD_DST, jnp.int32)]).reshape(-1, K)
    bidx = jnp.concatenate(
        [jnp.asarray(batch, jnp.int32),
         jnp.full((NPAD - N_NODES,), PAD_GRAPH, jnp.int32)]).reshape(-1, K)

    xpad = jnp.concatenate(
        [x, jnp.zeros((NPAD - N_NODES, D), jnp.float32)], axis=0)
    ones = jnp.ones((K, CW), jnp.float32)
    zrows = jnp.zeros((K, D), jnp.float32)
    zcol = jnp.zeros((K, CW), jnp.float32)
    zcol_g = jnp.zeros((GPAD, CW), jnp.float32)
    bl1r, bl2r, bl3r = (v.reshape(1, D) for v in (bl1, bl2, bl3))

    cnt, = _cnt_kernel(dst, ones, zcol)
    agg1, = _agg_kernel(xpad, src, dst, zrows)
    h1 = _dense_layer(agg1, cnt, xpad, Wl1, bl1r, Wr1, relu=True)
    agg2, = _agg_kernel(h1, src, dst, zrows)
    h2 = _dense_layer(agg2, cnt, h1, Wl2, bl2r, Wr2, relu=True)
    agg3, = _agg_kernel(h2, src, dst, zrows)
    h3 = _dense_layer(agg3, cnt, h2, Wl3, bl3r, Wr3, relu=False)
    pool, pcnt = _pool_kernel(h3, bidx, ones, zrows, zcol_g)
    return _final_layer(pool, pcnt, W, b.reshape(1, -1))


# speedup vs baseline: 1.5552x; 1.0615x over previous
import functools

import jax
import jax.numpy as jnp
from jax import lax
from jax.experimental import pallas as pl
from jax.experimental.pallas import tpu as pltpu
from jax.experimental.pallas import tpu_sc as plsc

N_NODES = 10000
N_GRAPHS = 64
D = 128

NPAD = 10240
EPAD = 327680
K = 128
CHUNKS_PER_TILE = 80
ROWS_PER_TILE = 640
GPAD = 128
BCHUNKS = 80
PAD_DST = NPAD - 8
PAD_GRAPH = N_GRAPHS
CW = 16

_MESH = plsc.VectorSubcoreMesh(core_axis_name="c", subcore_axis_name="s")
_SC_PARAMS = pltpu.CompilerParams(use_tc_tiling_on_sc=False)


def _zero_my_rows(zrows_hbm, rows_v, table_s, r0, nrows):
    pltpu.sync_copy(zrows_hbm, rows_v)
    @pl.loop(0, nrows // K)
    def _(j):
        pltpu.sync_copy(rows_v, table_s.at[pl.ds(r0 + j * K, K)])


_G = 16
_NG = [9, 1]
assert 16 * (_NG[0] + _NG[1]) * _G == EPAD // K


def _make_agg_kernel():
    outs = [jax.ShapeDtypeStruct((2 * NPAD, D), jnp.float32)]
    scratch = [
        pltpu.VMEM((_G, K), jnp.int32),
        pltpu.VMEM((_G, K), jnp.int32),
        pltpu.VMEM((K, D), jnp.float32),
        pltpu.VMEM((K, D), jnp.float32),
        pltpu.VMEM_SHARED((NPAD, D), jnp.float32),
        pltpu.SemaphoreType.DMA,
        pltpu.SemaphoreType.DMA,
        pltpu.SemaphoreType.DMA,
        pltpu.SemaphoreType.DMA,
    ]

    def body(h_hbm, src_hbm, dst_hbm, zrows_hbm, agg_hbm,
             src_v, dst_v, rows0, rows1, agg_s, g0, g1, s0, s1):
        cid = lax.axis_index("c")
        sid = lax.axis_index("s")
        r0 = sid * ROWS_PER_TILE

        _zero_my_rows(zrows_hbm, rows0, agg_s, r0, ROWS_PER_TILE)
        ngroups = jnp.where(cid == 0, _NG[0], _NG[1])
        c0 = jnp.where(cid == 0, sid * (_NG[0] * _G),
                       16 * _NG[0] * _G + sid * (_NG[1] * _G))
        plsc.subcore_barrier()

        def gath(i, buf, sem):
            pltpu.async_copy(h_hbm.at[src_v.at[i]], buf, sem)

        def wait_g(i, buf, sem):
            pltpu.make_async_copy(h_hbm.at[src_v.at[i]], buf, sem).wait()

        def scat(i, buf, sem):
            pltpu.async_copy(buf, agg_s.at[dst_v.at[i]], sem, add=True)

        def wait_s(i, buf, sem):
            pltpu.make_async_copy(
                buf, agg_s.at[dst_v.at[i]], sem).wait()

        @pl.loop(0, ngroups)
        def _(grp):
            h0 = c0 + grp * _G
            pltpu.sync_copy(src_hbm.at[pl.ds(h0, _G)], src_v)
            pltpu.sync_copy(dst_hbm.at[pl.ds(h0, _G)], dst_v)
            gath(0, rows0, g0)
            @pl.loop(0, _G // 2)
            def _(j):
                i0 = 2 * j
                i1 = 2 * j + 1
                wait_g(i0, rows0, g0)
                scat(i0, rows0, s0)
                @pl.when(j > 0)
                def _():
                    wait_s(i1, rows1, s1)
                gath(i1, rows1, g1)
                wait_g(i1, rows1, g1)
                scat(i1, rows1, s1)
                wait_s(i0, rows0, s0)
                @pl.when(j < _G // 2 - 1)
                def _():
                    gath(i0 + 2, rows0, g0)
            wait_s(_G - 1, rows1, s1)

        plsc.subcore_barrier()
        out0 = cid * NPAD + r0
        @pl.loop(0, ROWS_PER_TILE // K)
        def _(j):
            pltpu.sync_copy(agg_s.at[pl.ds(r0 + j * K, K)], rows0)
            pltpu.sync_copy(rows0, agg_hbm.at[pl.ds(out0 + j * K, K)])

    return pl.kernel(body, out_type=outs, mesh=_MESH, scratch_types=scratch,
                     compiler_params=_SC_PARAMS, name="sc_agg")


_agg_kernel = _make_agg_kernel()


_HALF = CHUNKS_PER_TILE // 2


def _cnt_body(dst_hbm, ones_hbm, zcol_hbm, cnt_hbm,
              dst_v, ones_v, col_v, cnt_s):
    cid = lax.axis_index("c")
    sid = lax.axis_index("s")
    wid = sid * 2 + cid
    r0 = sid * ROWS_PER_TILE

    pltpu.sync_copy(zcol_hbm, col_v)
    @pl.loop(0, ROWS_PER_TILE // K)
    def _(j):
        pltpu.sync_copy(col_v, cnt_s.at[pl.ds(r0 + j * K, K)])
    pltpu.sync_copy(ones_hbm, ones_v)
    c0 = wid * CHUNKS_PER_TILE
    plsc.subcore_barrier()

    @pl.loop(0, 2)
    def _(half):
        pltpu.sync_copy(dst_hbm.at[pl.ds(c0 + half * _HALF, _HALF)], dst_v)
        @pl.loop(0, _HALF)
        def _(i):
            pltpu.sync_copy(ones_v, cnt_s.at[dst_v.at[i]], add=True)

    plsc.subcore_barrier()
    out0 = cid * NPAD + r0
    @pl.loop(0, ROWS_PER_TILE // K)
    def _(j):
        pltpu.sync_copy(cnt_s.at[pl.ds(r0 + j * K, K)], col_v)
        pltpu.sync_copy(col_v, cnt_hbm.at[pl.ds(out0 + j * K, K)])


_cnt_kernel = pl.kernel(
    _cnt_body,
    out_type=[jax.ShapeDtypeStruct((2 * NPAD, CW), jnp.float32)],
    mesh=_MESH,
    scratch_types=[
        pltpu.VMEM((_HALF, K), jnp.int32),
        pltpu.VMEM((K, CW), jnp.float32),
        pltpu.VMEM((K, CW), jnp.float32),
        pltpu.VMEM_SHARED((NPAD, CW), jnp.float32),
    ],
    compiler_params=_SC_PARAMS,
    name="sc_cnt",
)


def _pool_body(h_hbm, bidx_hbm, ones_hbm, zrows_hbm, zcol_hbm,
               pool_hbm, pcnt_hbm, bidx_v, rows_v, ones_v, col_v,
               pool_s, pcnt_s, sem):
    cid = lax.axis_index("c")
    sid = lax.axis_index("s")
    wid = sid * 2 + cid

    @pl.when(sid == 0)
    def _():
        _zero_my_rows(zrows_hbm, rows_v, pool_s, 0, GPAD)
        pltpu.sync_copy(zcol_hbm, col_v)
        pltpu.sync_copy(col_v, pcnt_s)
    pltpu.sync_copy(ones_hbm, ones_v)
    pltpu.sync_copy(bidx_hbm.at[pl.ds(0, BCHUNKS)], bidx_v)
    plsc.subcore_barrier()

    @pl.loop(0, 3)
    def _(j):
        chunk = wid + j * 32
        @pl.when(chunk < BCHUNKS)
        def _():
            pltpu.async_copy(h_hbm.at[pl.ds(chunk * K, K)], rows_v, sem).wait()
            pltpu.sync_copy(rows_v, pool_s.at[bidx_v.at[chunk]], add=True)
            pltpu.sync_copy(ones_v, pcnt_s.at[bidx_v.at[chunk]], add=True)

    plsc.subcore_barrier()
    @pl.when(sid == 0)
    def _():
        pltpu.sync_copy(pool_s, rows_v)
        pltpu.sync_copy(rows_v, pool_hbm.at[pl.ds(cid * GPAD, GPAD)])
        pltpu.sync_copy(pcnt_s, col_v)
        pltpu.sync_copy(col_v, pcnt_hbm.at[pl.ds(cid * GPAD, GPAD)])


_pool_kernel = pl.kernel(
    _pool_body,
    out_type=[jax.ShapeDtypeStruct((2 * GPAD, D), jnp.float32),
              jax.ShapeDtypeStruct((2 * GPAD, CW), jnp.float32)],
    mesh=_MESH,
    scratch_types=[
        pltpu.VMEM((BCHUNKS, K), jnp.int32),
        pltpu.VMEM((K, D), jnp.float32),
        pltpu.VMEM((K, CW), jnp.float32),
        pltpu.VMEM((GPAD, CW), jnp.float32),
        pltpu.VMEM_SHARED((GPAD, D), jnp.float32),
        pltpu.VMEM_SHARED((GPAD, CW), jnp.float32),
        pltpu.SemaphoreType.DMA,
    ],
    compiler_params=_SC_PARAMS,
    name="sc_pool",
)


_DOT = functools.partial(
    lax.dot_general,
    precision=lax.Precision.HIGHEST,
    preferred_element_type=jnp.float32,
)

_RBLK = 640
_RGRID = NPAD // _RBLK


def _dense_body(relu, aggA, aggB, cA, cB, h, Wl, Wr, bl, o):
    cnt = jnp.maximum(cA[...] + cB[...], 1.0)[:, :1]
    mean = (aggA[...] + aggB[...]) / cnt
    r = _DOT(mean, Wl[...], (((1,), (1,)), ((), ())))
    r = r + _DOT(h[...], Wr[...], (((1,), (1,)), ((), ())))
    r = r + bl[...]
    o[...] = jnp.maximum(r, 0.0) if relu else r


def _dense_layer(agg, cnt, h, Wl, bl, Wr, relu):
    body = functools.partial(_dense_body, relu)
    return pl.pallas_call(
        body,
        grid=(_RGRID,),
        in_specs=[
            pl.BlockSpec((_RBLK, D), lambda i: (i, 0)),
            pl.BlockSpec((_RBLK, D), lambda i: (i + _RGRID, 0)),
            pl.BlockSpec((_RBLK, CW), lambda i: (i, 0)),
            pl.BlockSpec((_RBLK, CW), lambda i: (i + _RGRID, 0)),
            pl.BlockSpec((_RBLK, D), lambda i: (i, 0)),
            pl.BlockSpec((D, D), lambda i: (0, 0)),
            pl.BlockSpec((D, D), lambda i: (0, 0)),
            pl.BlockSpec((1, D), lambda i: (0, 0)),
        ],
        out_specs=pl.BlockSpec((_RBLK, D), lambda i: (i, 0)),
        out_shape=jax.ShapeDtypeStruct((NPAD, D), jnp.float32),
    )(agg, agg, cnt, cnt, h, Wl, Wr, bl)


def _final_body(pA, pB, cA, cB, W, b, o):
    cnt = jnp.maximum(cA[...] + cB[...], 1.0)[:, :1]
    g = (pA[...] + pB[...]) / cnt
    o[...] = _DOT(g, W[...], (((1,), (1,)), ((), ()))) + b[...]


def _final_layer(pool, pcnt, W, b):
    nb = GPAD // N_GRAPHS
    return pl.pallas_call(
        _final_body,
        grid=(1,),
        in_specs=[
            pl.BlockSpec((N_GRAPHS, D), lambda i: (0, 0)),
            pl.BlockSpec((N_GRAPHS, D), lambda i: (nb, 0)),
            pl.BlockSpec((N_GRAPHS, CW), lambda i: (0, 0)),
            pl.BlockSpec((N_GRAPHS, CW), lambda i: (nb, 0)),
            pl.BlockSpec(W.shape, lambda i: (0, 0)),
            pl.BlockSpec((1, W.shape[0]), lambda i: (0, 0)),
        ],
        out_specs=pl.BlockSpec((N_GRAPHS, W.shape[0]), lambda i: (0, 0)),
        out_shape=jax.ShapeDtypeStruct((N_GRAPHS, W.shape[0]), jnp.float32),
    )(pool, pool, pcnt, pcnt, W, b)


def kernel(x, edge_index, batch, Wl1, bl1, Wr1, Wl2, bl2, Wr2, Wl3, bl3,
           Wr3, W, b):
    e = edge_index.shape[1]
    src = jnp.asarray(edge_index[0], jnp.int32)
    dst = jnp.asarray(edge_index[1], jnp.int32)
    src = jnp.concatenate(
        [src, jnp.zeros((EPAD - e,), jnp.int32)]).reshape(-1, K)
    dst = jnp.concatenate(
        [dst, jnp.full((EPAD - e,), PAD_DST, jnp.int32)]).reshape(-1, K)
    bidx = jnp.concatenate(
        [jnp.asarray(batch, jnp.int32),
         jnp.full((NPAD - N_NODES,), PAD_GRAPH, jnp.int32)]).reshape(-1, K)

    xpad = jnp.concatenate(
        [x, jnp.zeros((NPAD - N_NODES, D), jnp.float32)], axis=0)
    ones = jnp.ones((K, CW), jnp.float32)
    zrows = jnp.zeros((K, D), jnp.float32)
    zcol = jnp.zeros((K, CW), jnp.float32)
    zcol_g = jnp.zeros((GPAD, CW), jnp.float32)
    bl1r, bl2r, bl3r = (v.reshape(1, D) for v in (bl1, bl2, bl3))

    cnt, = _cnt_kernel(dst, ones, zcol)
    agg1, = _agg_kernel(xpad, src, dst, zrows)
    h1 = _dense_layer(agg1, cnt, xpad, Wl1, bl1r, Wr1, relu=True)
    agg2, = _agg_kernel(h1, src, dst, zrows)
    h2 = _dense_layer(agg2, cnt, h1, Wl2, bl2r, Wr2, relu=True)
    agg3, = _agg_kernel(h2, src, dst, zrows)
    h3 = _dense_layer(agg3, cnt, h2, Wl3, bl3r, Wr3, relu=False)
    pool, pcnt = _pool_kernel(h3, bidx, ones, zrows, zcol_g)
    return _final_layer(pool, pcnt, W, b.reshape(1, -1))
